# baseline (speedup 1.0000x reference)
import functools

import jax
import jax.numpy as jnp
from jax import lax
from jax.experimental import pallas as pl
from jax.experimental.pallas import tpu as pltpu
from jax.experimental.pallas import tpu_sc as plsc

_D = 128
_V = 300
_VP = 304
_NC = 2
_NS = 16
_NW = _NC * _NS
_CH = 256
_KI = 10
_NBUF = 2
_P = 1
_NG8 = _VP // 8
_LANE = 16


@functools.partial(jax.jit, static_argnames=("n_total",))
def _vqae_sc(idx, W, n_total):
    nb = n_total // _NW
    nch = nb // _CH
    ng = nch // _KI
    assert ng * _KI == nch and nch * _CH == nb and ng % 2 == 0

    mesh = plsc.VectorSubcoreMesh(
        core_axis_name="c", subcore_axis_name="s",
        num_cores=_NC, num_subcores=_NS,
    )

    scratch = (
        [pltpu.VMEM((_KI * _CH,), jnp.int32) for _ in range(2)]
        + [pltpu.VMEM_SHARED((_V, _D), jnp.float32)]
        + [pltpu.VMEM((_CH, _D), jnp.float32) for _ in range(_NBUF)]
        + [pltpu.SemaphoreType.DMA for _ in range(2 * _NBUF + 2)]
    )

    @functools.partial(
        pl.kernel,
        mesh=mesh,
        out_type=jax.ShapeDtypeStruct((n_total, _D), jnp.float32),
        scratch_types=scratch,
    )
    def k(idx_hbm, tab_hbm, out_hbm, ib0, ib1, tab_v,
          r0, r1, g0, g1, s0, s1,
          i0, i1):
        rows = (r0, r1)
        gsem = (g0, g1)
        ssem = (s0, s1)
        ibuf = (ib0, ib1)
        isem = (i0, i1)
        cid = lax.axis_index("c")
        sid = lax.axis_index("s")
        wid = sid * _NC + cid
        row0 = wid * nb

        def iload(h, p):
            return pltpu.make_async_copy(
                idx_hbm.at[pl.ds(row0 + h * (_KI * _CH), _KI * _CH)],
                ibuf[p], isem[p])

        def gat(idx_v, j, buf):
            return pltpu.make_async_copy(
                tab_v.at[idx_v.at[pl.ds(j * _CH, _CH)]], rows[buf],
                gsem[buf])

        iload(0, 0).start()
        iload(1, 1).start()

        @pl.when(sid == 0)
        def _():
            pltpu.sync_copy(tab_hbm, tab_v)
        plsc.subcore_barrier()

        def run_group(h, p, first):
            idx_v = ibuf[p]
            gbase = row0 + h * (_KI * _CH)
            iload(h, p).wait()
            for j in range(_KI):
                b = j % _NBUF
                if j >= _NBUF:
                    pltpu.make_async_copy(
                        rows[b], out_hbm.at[pl.ds(gbase + j * _CH, _CH)],
                        ssem[b]).wait()
                elif first is not None:
                    @pl.when(jnp.logical_not(first))
                    def _():
                        pltpu.make_async_copy(
                            rows[b], out_hbm.at[pl.ds(gbase + j * _CH, _CH)],
                            ssem[b]).wait()
                else:
                    pltpu.make_async_copy(
                        rows[b], out_hbm.at[pl.ds(gbase + j * _CH, _CH)],
                        ssem[b]).wait()
                gat(idx_v, j, b).start()
                jj = j - _P
                if jj >= 0:
                    bb = jj % _NBUF
                    gat(idx_v, jj, bb).wait()
                    pltpu.async_copy(
                        rows[bb], out_hbm.at[pl.ds(gbase + jj * _CH, _CH)],
                        ssem[bb])
            for jj in range(_KI - _P, _KI):
                bb = jj % _NBUF
                gat(idx_v, jj, bb).wait()
                pltpu.async_copy(
                    rows[bb], out_hbm.at[pl.ds(gbase + jj * _CH, _CH)],
                    ssem[bb])
            @pl.when(h + 2 < ng)
            def _():
                iload(h + 2, p).start()

        def pair(t, carry):
            run_group(2 * t, 0, t == 0)
            run_group(2 * t + 1, 1, None)
            return carry

        lax.fori_loop(0, ng // 2, pair, 0)

        for j in range(_KI - _NBUF, _KI):
            b = j % _NBUF
            cbase = row0 + (ng - 1) * (_KI * _CH) + j * _CH
            pltpu.make_async_copy(
                rows[b], out_hbm.at[pl.ds(cbase, _CH)], ssem[b]).wait()

    return k(idx, W)


def _normalize_table(W):
    def body(w_ref, o_ref):
        w = w_ref[...]
        ss = jnp.sum(w * w, axis=-1, keepdims=True)
        o_ref[...] = w / (jnp.sqrt(ss) + 1e-6)

    return pl.pallas_call(
        body,
        out_shape=jax.ShapeDtypeStruct(W.shape, W.dtype),
    )(W)


def kernel(code, W):
    n_total = code.shape[0] * code.shape[1]
    idx = code.reshape(-1).astype(jnp.int32)
    Wn = _normalize_table(W.astype(jnp.float32))
    out = _vqae_sc(idx, Wn, n_total)
    return out.reshape(*code.shape, _D)


# speedup vs baseline: 1.0241x; 1.0241x over previous
import functools

import jax
import jax.numpy as jnp
from jax import lax
from jax.experimental import pallas as pl
from jax.experimental.pallas import tpu as pltpu
from jax.experimental.pallas import tpu_sc as plsc

_D = 128
_V = 300
_VP = 304
_NC = 2
_NS = 16
_NW = _NC * _NS
_CH = 128
_KI = 40
_NBUF = 5
_P = 3
_NG8 = _VP // 8
_LANE = 16


@functools.partial(jax.jit, static_argnames=("n_total",))
def _vqae_sc(idx, W, n_total):
    nb = n_total // _NW
    nch = nb // _CH
    ng = nch // _KI
    assert ng * _KI == nch and nch * _CH == nb and ng % 2 == 0

    mesh = plsc.VectorSubcoreMesh(
        core_axis_name="c", subcore_axis_name="s",
        num_cores=_NC, num_subcores=_NS,
    )

    scratch = (
        [pltpu.VMEM((_KI * _CH,), jnp.int32) for _ in range(2)]
        + [pltpu.VMEM_SHARED((_V, _D), jnp.float32)]
        + [pltpu.VMEM((_CH, _D), jnp.float32) for _ in range(_NBUF)]
        + [pltpu.SemaphoreType.DMA for _ in range(2 * _NBUF + 2)]
    )

    @functools.partial(
        pl.kernel,
        mesh=mesh,
        out_type=jax.ShapeDtypeStruct((n_total, _D), jnp.float32),
        scratch_types=scratch,
    )
    def k(idx_hbm, tab_hbm, out_hbm, ib0, ib1, tab_v,
          r0, r1, r2, r3, r4, g0, g1, g2, g3, g4, s0, s1, s2, s3, s4,
          i0, i1):
        rows = (r0, r1, r2, r3, r4)
        gsem = (g0, g1, g2, g3, g4)
        ssem = (s0, s1, s2, s3, s4)
        ibuf = (ib0, ib1)
        isem = (i0, i1)
        cid = lax.axis_index("c")
        sid = lax.axis_index("s")
        wid = sid * _NC + cid
        row0 = wid * nb

        def iload(h, p):
            return pltpu.make_async_copy(
                idx_hbm.at[pl.ds(row0 + h * (_KI * _CH), _KI * _CH)],
                ibuf[p], isem[p])

        def gat(idx_v, j, buf):
            return pltpu.make_async_copy(
                tab_v.at[idx_v.at[pl.ds(j * _CH, _CH)]], rows[buf],
                gsem[buf])

        iload(0, 0).start()
        iload(1, 1).start()

        @pl.when(sid == 0)
        def _():
            pltpu.sync_copy(tab_hbm, tab_v)
        plsc.subcore_barrier()

        def run_group(h, p, first):
            idx_v = ibuf[p]
            gbase = row0 + h * (_KI * _CH)
            iload(h, p).wait()
            for j in range(_KI):
                b = j % _NBUF
                if j >= _NBUF:
                    pltpu.make_async_copy(
                        rows[b], out_hbm.at[pl.ds(gbase + j * _CH, _CH)],
                        ssem[b]).wait()
                elif first is not None:
                    @pl.when(jnp.logical_not(first))
                    def _():
                        pltpu.make_async_copy(
                            rows[b], out_hbm.at[pl.ds(gbase + j * _CH, _CH)],
                            ssem[b]).wait()
                else:
                    pltpu.make_async_copy(
                        rows[b], out_hbm.at[pl.ds(gbase + j * _CH, _CH)],
                        ssem[b]).wait()
                gat(idx_v, j, b).start()
                jj = j - _P
                if jj >= 0:
                    bb = jj % _NBUF
                    gat(idx_v, jj, bb).wait()
                    pltpu.async_copy(
                        rows[bb], out_hbm.at[pl.ds(gbase + jj * _CH, _CH)],
                        ssem[bb])
            for jj in range(_KI - _P, _KI):
                bb = jj % _NBUF
                gat(idx_v, jj, bb).wait()
                pltpu.async_copy(
                    rows[bb], out_hbm.at[pl.ds(gbase + jj * _CH, _CH)],
                    ssem[bb])
            @pl.when(h + 2 < ng)
            def _():
                iload(h + 2, p).start()

        def pair(t, carry):
            run_group(2 * t, 0, t == 0)
            run_group(2 * t + 1, 1, None)
            return carry

        lax.fori_loop(0, ng // 2, pair, 0)

        for j in range(_KI - _NBUF, _KI):
            b = j % _NBUF
            cbase = row0 + (ng - 1) * (_KI * _CH) + j * _CH
            pltpu.make_async_copy(
                rows[b], out_hbm.at[pl.ds(cbase, _CH)], ssem[b]).wait()

    return k(idx, W)


def _normalize_table(W):
    def body(w_ref, o_ref):
        w = w_ref[...]
        ss = jnp.sum(w * w, axis=-1, keepdims=True)
        o_ref[...] = w / (jnp.sqrt(ss) + 1e-6)

    return pl.pallas_call(
        body,
        out_shape=jax.ShapeDtypeStruct(W.shape, W.dtype),
    )(W)


def kernel(code, W):
    n_total = code.shape[0] * code.shape[1]
    idx = code.reshape(-1).astype(jnp.int32)
    Wn = _normalize_table(W.astype(jnp.float32))
    out = _vqae_sc(idx, Wn, n_total)
    return out.reshape(*code.shape, _D)
